# baseline (speedup 1.0000x reference)
import functools

import jax
import jax.numpy as jnp
from jax import lax
from jax.experimental import pallas as pl
from jax.experimental.pallas import tpu as pltpu
from jax.experimental.pallas import tpu_sc as plsc

N = 4096
EDIM = 128
WFEAT = 128
SLOPE = 0.01
RBLK = 256
NBLK = N // RBLK
NEG = -1e9


def _make_sc_gather(V, D, B):
    info = plsc.get_sparse_core_info()
    NC, NS = info.num_cores, info.num_subcores
    NW = NC * NS
    assert B % (8 * NW) == 0 and D % info.num_lanes == 0
    b_per_w = B // NW
    mesh = plsc.VectorSubcoreMesh(core_axis_name="c", subcore_axis_name="s")

    @functools.partial(
        pl.kernel,
        mesh=mesh,
        out_type=jax.ShapeDtypeStruct((B, D), jnp.float32),
        scratch_types=[
            pltpu.VMEM((b_per_w,), jnp.int32),
            pltpu.VMEM((b_per_w, D), jnp.float32),
            pltpu.SemaphoreType.DMA,
        ],
    )
    def gather_k(idx_hbm, table_hbm, out_hbm, idx_v, rows_v, sem):
        wid = lax.axis_index("s") * NC + lax.axis_index("c")
        base = wid * b_per_w
        pltpu.sync_copy(idx_hbm.at[pl.ds(base, b_per_w)], idx_v)
        pltpu.async_copy(table_hbm.at[idx_v], rows_v, sem).wait()
        pltpu.sync_copy(rows_v, out_hbm.at[pl.ds(base, b_per_w)])

    return gather_k


@functools.lru_cache(maxsize=1)
def _sc_gather_fn():
    return _make_sc_gather(100000, EDIM, N)


def _sc_gather(idx, table):
    return _sc_gather_fn()(idx, table)


def _pre_body(x_ref, w_ref, a1_ref, a2_ref, h_ref, f1_ref, f2_ref):
    h = jnp.dot(x_ref[...], w_ref[...], preferred_element_type=jnp.float32)
    h_ref[...] = h
    f1_ref[...] = jnp.dot(h, a1_ref[...], preferred_element_type=jnp.float32)
    f2_ref[...] = lax.dot_general(
        a2_ref[...], h, (((1,), (1,)), ((), ())),
        preferred_element_type=jnp.float32)


def _pre(x, W, a1, a2):
    return pl.pallas_call(
        _pre_body,
        out_shape=(
            jax.ShapeDtypeStruct((N, WFEAT), jnp.float32),
            jax.ShapeDtypeStruct((N, 1), jnp.float32),
            jax.ShapeDtypeStruct((1, N), jnp.float32),
        ),
    )(x, W, a1, a2)


def _att_scores(adj, f1, f2row):
    s = f1 + f2row
    e = jnp.where(s >= 0, s, SLOPE * s)
    p = jnp.exp(jnp.where(adj.astype(jnp.int32) != 0, e, NEG))
    return p / jnp.sum(p, axis=1, keepdims=True)


def _att1_body(adj_ref, f1_ref, f2_ref, h_ref, att_ref, x2_ref):
    att = _att_scores(adj_ref[...], f1_ref[...], f2_ref[...])
    att_ref[...] = att
    out = jnp.dot(att.astype(jnp.bfloat16), h_ref[...].astype(jnp.bfloat16),
                  preferred_element_type=jnp.float32)
    x2_ref[...] = jnp.where(out > 0, out, (jnp.exp(out) - 1.0))


def _att2_body(adj_ref, f1_ref, f2_ref, h_ref, xres_ref, att_ref, dsum_ref):
    att = _att_scores(adj_ref[...], f1_ref[...], f2_ref[...])
    att_ref[...] = att
    out = jnp.dot(att.astype(jnp.bfloat16), h_ref[...].astype(jnp.bfloat16),
                  preferred_element_type=jnp.float32)
    doc = jnp.where(out > 0, out, (jnp.exp(out) - 1.0)) + xres_ref[...]
    part = jnp.sum(doc, axis=0, keepdims=True)

    @pl.when(pl.program_id(0) == 0)
    def _():
        dsum_ref[...] = part

    @pl.when(pl.program_id(0) > 0)
    def _():
        dsum_ref[...] += part


_ROWBLK = pl.BlockSpec((RBLK, N), lambda i: (i, 0))
_PKBLK = pl.BlockSpec((RBLK // 8, N), lambda i: (i, 0))
_F1BLK = pl.BlockSpec((RBLK, 1), lambda i: (i, 0))
_FULL_F2 = pl.BlockSpec((1, N), lambda i: (0, 0))
_FULL_H = pl.BlockSpec((N, WFEAT), lambda i: (0, 0))
_XBLK = pl.BlockSpec((RBLK, WFEAT), lambda i: (i, 0))
_ACC = pl.BlockSpec((1, WFEAT), lambda i: (0, 0))


def _att_layer1(adj, f1, f2row, h):
    return pl.pallas_call(
        _att1_body,
        grid=(NBLK,),
        in_specs=[_ROWBLK, _F1BLK, _FULL_F2, _FULL_H],
        out_specs=(_ROWBLK, _XBLK),
        out_shape=(
            jax.ShapeDtypeStruct((N, N), jnp.float32),
            jax.ShapeDtypeStruct((N, WFEAT), jnp.float32),
        ),
        compiler_params=pltpu.CompilerParams(
            dimension_semantics=("arbitrary",)),
    )(adj, f1, f2row, h)


def _att_layer2(adj, f1, f2row, h, xres):
    return pl.pallas_call(
        _att2_body,
        grid=(NBLK,),
        in_specs=[_ROWBLK, _F1BLK, _FULL_F2, _FULL_H, _XBLK],
        out_specs=(_ROWBLK, _ACC),
        out_shape=(
            jax.ShapeDtypeStruct((N, N), jnp.float32),
            jax.ShapeDtypeStruct((1, WFEAT), jnp.float32),
        ),
        compiler_params=pltpu.CompilerParams(
            dimension_semantics=("arbitrary",)),
    )(adj, f1, f2row, h, xres)


def kernel(inDoc, adj0, adj1, emb, W_s, a_s, W_d, a_d):
    words = _sc_gather(inDoc.astype(jnp.int32), emb)

    a1s = a_s[:WFEAT].reshape(WFEAT, 1)
    a2s = a_s[WFEAT:].reshape(1, WFEAT)
    h1, f1, f2r = _pre(words, W_s, a1s, a2s)
    satt, x2 = _att_layer1(adj0.astype(jnp.int4), f1, f2r, h1)

    a1d = a_d[:WFEAT].reshape(WFEAT, 1)
    a2d = a_d[WFEAT:].reshape(1, WFEAT)
    h2, g1, g2r = _pre(x2, W_d, a1d, a2d)
    datt, dsum = _att_layer2(adj1.astype(jnp.int4), g1, g2r, h2, x2)

    docMean = (dsum / jnp.float32(N)).reshape(WFEAT)
    return (docMean, satt, datt)


# speedup vs baseline: 1.0057x; 1.0057x over previous
import functools

import jax
import jax.numpy as jnp
from jax import lax
from jax.experimental import pallas as pl
from jax.experimental.pallas import tpu as pltpu
from jax.experimental.pallas import tpu_sc as plsc

N = 4096
EDIM = 128
WFEAT = 128
SLOPE = 0.01
RBLK = 512
NBLK = N // RBLK
NEG = -1e9


def _make_sc_gather(V, D, B):
    info = plsc.get_sparse_core_info()
    NC, NS = info.num_cores, info.num_subcores
    NW = NC * NS
    assert B % (8 * NW) == 0 and D % info.num_lanes == 0
    b_per_w = B // NW
    mesh = plsc.VectorSubcoreMesh(core_axis_name="c", subcore_axis_name="s")

    @functools.partial(
        pl.kernel,
        mesh=mesh,
        out_type=jax.ShapeDtypeStruct((B, D), jnp.float32),
        scratch_types=[
            pltpu.VMEM((b_per_w,), jnp.int32),
            pltpu.VMEM((b_per_w, D), jnp.float32),
            pltpu.SemaphoreType.DMA,
        ],
    )
    def gather_k(idx_hbm, table_hbm, out_hbm, idx_v, rows_v, sem):
        wid = lax.axis_index("s") * NC + lax.axis_index("c")
        base = wid * b_per_w
        pltpu.sync_copy(idx_hbm.at[pl.ds(base, b_per_w)], idx_v)
        pltpu.async_copy(table_hbm.at[idx_v], rows_v, sem).wait()
        pltpu.sync_copy(rows_v, out_hbm.at[pl.ds(base, b_per_w)])

    return gather_k


@functools.lru_cache(maxsize=1)
def _sc_gather_fn():
    return _make_sc_gather(100000, EDIM, N)


def _sc_gather(idx, table):
    return _sc_gather_fn()(idx, table)


def _pre_body(x_ref, w_ref, a1_ref, a2_ref, h_ref, f1_ref, f2_ref):
    h = jnp.dot(x_ref[...], w_ref[...], preferred_element_type=jnp.float32)
    h_ref[...] = h
    f1_ref[...] = jnp.dot(h, a1_ref[...], preferred_element_type=jnp.float32)
    f2_ref[...] = lax.dot_general(
        a2_ref[...], h, (((1,), (1,)), ((), ())),
        preferred_element_type=jnp.float32)


def _pre(x, W, a1, a2):
    return pl.pallas_call(
        _pre_body,
        out_shape=(
            jax.ShapeDtypeStruct((N, WFEAT), jnp.float32),
            jax.ShapeDtypeStruct((N, 1), jnp.float32),
            jax.ShapeDtypeStruct((1, N), jnp.float32),
        ),
    )(x, W, a1, a2)


def _att_scores(adj, f1, f2row):
    s = f1 + f2row
    e = jnp.where(s >= 0, s, SLOPE * s)
    p = jnp.exp(jnp.where(adj.astype(jnp.int32) != 0, e, NEG))
    return p / jnp.sum(p, axis=1, keepdims=True)


def _att1_body(adj_ref, f1_ref, f2_ref, h_ref, att_ref, x2_ref):
    att = _att_scores(adj_ref[...], f1_ref[...], f2_ref[...])
    att_ref[...] = att
    out = jnp.dot(att.astype(jnp.bfloat16), h_ref[...].astype(jnp.bfloat16),
                  preferred_element_type=jnp.float32)
    x2_ref[...] = jnp.where(out > 0, out, (jnp.exp(out) - 1.0))


def _att2_body(adj_ref, f1_ref, f2_ref, h_ref, xres_ref, att_ref, dsum_ref):
    att = _att_scores(adj_ref[...], f1_ref[...], f2_ref[...])
    att_ref[...] = att
    out = jnp.dot(att.astype(jnp.bfloat16), h_ref[...].astype(jnp.bfloat16),
                  preferred_element_type=jnp.float32)
    doc = jnp.where(out > 0, out, (jnp.exp(out) - 1.0)) + xres_ref[...]
    part = jnp.sum(doc, axis=0, keepdims=True)

    @pl.when(pl.program_id(0) == 0)
    def _():
        dsum_ref[...] = part

    @pl.when(pl.program_id(0) > 0)
    def _():
        dsum_ref[...] += part


_ROWBLK = pl.BlockSpec((RBLK, N), lambda i: (i, 0))
_PKBLK = pl.BlockSpec((RBLK // 8, N), lambda i: (i, 0))
_F1BLK = pl.BlockSpec((RBLK, 1), lambda i: (i, 0))
_FULL_F2 = pl.BlockSpec((1, N), lambda i: (0, 0))
_FULL_H = pl.BlockSpec((N, WFEAT), lambda i: (0, 0))
_XBLK = pl.BlockSpec((RBLK, WFEAT), lambda i: (i, 0))
_ACC = pl.BlockSpec((1, WFEAT), lambda i: (0, 0))


def _att_layer1(adj, f1, f2row, h):
    return pl.pallas_call(
        _att1_body,
        grid=(NBLK,),
        in_specs=[_ROWBLK, _F1BLK, _FULL_F2, _FULL_H],
        out_specs=(_ROWBLK, _XBLK),
        out_shape=(
            jax.ShapeDtypeStruct((N, N), jnp.float32),
            jax.ShapeDtypeStruct((N, WFEAT), jnp.float32),
        ),
        compiler_params=pltpu.CompilerParams(
            dimension_semantics=("arbitrary",)),
    )(adj, f1, f2row, h)


def _att_layer2(adj, f1, f2row, h, xres):
    return pl.pallas_call(
        _att2_body,
        grid=(NBLK,),
        in_specs=[_ROWBLK, _F1BLK, _FULL_F2, _FULL_H, _XBLK],
        out_specs=(_ROWBLK, _ACC),
        out_shape=(
            jax.ShapeDtypeStruct((N, N), jnp.float32),
            jax.ShapeDtypeStruct((1, WFEAT), jnp.float32),
        ),
        compiler_params=pltpu.CompilerParams(
            dimension_semantics=("arbitrary",)),
    )(adj, f1, f2row, h, xres)


def kernel(inDoc, adj0, adj1, emb, W_s, a_s, W_d, a_d):
    words = _sc_gather(inDoc.astype(jnp.int32), emb)

    a1s = a_s[:WFEAT].reshape(WFEAT, 1)
    a2s = a_s[WFEAT:].reshape(1, WFEAT)
    h1, f1, f2r = _pre(words, W_s, a1s, a2s)
    satt, x2 = _att_layer1(adj0.astype(jnp.int2), f1, f2r, h1)

    a1d = a_d[:WFEAT].reshape(WFEAT, 1)
    a2d = a_d[WFEAT:].reshape(1, WFEAT)
    h2, g1, g2r = _pre(x2, W_d, a1d, a2d)
    datt, dsum = _att_layer2(adj1.astype(jnp.int2), g1, g2r, h2, x2)

    docMean = (dsum / jnp.float32(N)).reshape(WFEAT)
    return (docMean, satt, datt)


# speedup vs baseline: 1.0903x; 1.0841x over previous
import functools

import jax
import jax.numpy as jnp
from jax import lax
from jax.experimental import pallas as pl
from jax.experimental.pallas import tpu as pltpu
from jax.experimental.pallas import tpu_sc as plsc

N = 4096
EDIM = 128
WFEAT = 128
SLOPE = 0.01
RBLK = 512
NBLK = N // RBLK
NEG = -1e9


def _make_sc_gather(V, D, B):
    info = plsc.get_sparse_core_info()
    NC, NS = info.num_cores, info.num_subcores
    NW = NC * NS
    assert B % (8 * NW) == 0 and D % info.num_lanes == 0
    b_per_w = B // NW
    mesh = plsc.VectorSubcoreMesh(core_axis_name="c", subcore_axis_name="s")

    @functools.partial(
        pl.kernel,
        mesh=mesh,
        out_type=jax.ShapeDtypeStruct((B, D), jnp.float32),
        scratch_types=[
            pltpu.VMEM((b_per_w,), jnp.int32),
            pltpu.VMEM((b_per_w, D), jnp.float32),
            pltpu.SemaphoreType.DMA,
        ],
    )
    def gather_k(idx_hbm, table_hbm, out_hbm, idx_v, rows_v, sem):
        wid = lax.axis_index("s") * NC + lax.axis_index("c")
        base = wid * b_per_w
        pltpu.sync_copy(idx_hbm.at[pl.ds(base, b_per_w)], idx_v)
        pltpu.async_copy(table_hbm.at[idx_v], rows_v, sem).wait()
        pltpu.sync_copy(rows_v, out_hbm.at[pl.ds(base, b_per_w)])

    return gather_k


@functools.lru_cache(maxsize=1)
def _sc_gather_fn():
    return _make_sc_gather(100000, EDIM, N)


def _sc_gather(idx, table):
    return _sc_gather_fn()(idx, table)


def _project(x_ref, w_ref, a1_ref, a2_ref, h_sc, f1_sc, f2_sc):
    h = jnp.dot(x_ref[...], w_ref[...], preferred_element_type=jnp.float32)
    h_sc[...] = h
    f1_sc[...] = jnp.dot(h, a1_ref[...], preferred_element_type=jnp.float32)
    f2_sc[...] = lax.dot_general(
        a2_ref[...], h, (((1,), (1,)), ((), ())),
        preferred_element_type=jnp.float32)


def _att_scores(adj, f1, f2row):
    s = f1 + f2row
    e = jnp.where(s >= 0, s, SLOPE * s)
    p = jnp.exp(jnp.where(adj.astype(jnp.int32) != 0, e, NEG))
    return p / jnp.sum(p, axis=1, keepdims=True)


def _layer1_body(x_ref, w_ref, a1_ref, a2_ref, adj_ref, att_ref, x2_ref,
                 h_sc, f1_sc, f2_sc):
    i = pl.program_id(0)

    @pl.when(i == 0)
    def _():
        _project(x_ref, w_ref, a1_ref, a2_ref, h_sc, f1_sc, f2_sc)

    f1 = f1_sc[pl.ds(i * RBLK, RBLK), :]
    att = _att_scores(adj_ref[...], f1, f2_sc[...])
    att_ref[...] = att
    out = jnp.dot(att.astype(jnp.bfloat16), h_sc[...].astype(jnp.bfloat16),
                  preferred_element_type=jnp.float32)
    x2_ref[...] = jnp.where(out > 0, out, (jnp.exp(out) - 1.0))


def _layer2_body(x_ref, w_ref, a1_ref, a2_ref, adj_ref, att_ref, dsum_ref,
                 h_sc, f1_sc, f2_sc):
    i = pl.program_id(0)

    @pl.when(i == 0)
    def _():
        _project(x_ref, w_ref, a1_ref, a2_ref, h_sc, f1_sc, f2_sc)

    f1 = f1_sc[pl.ds(i * RBLK, RBLK), :]
    att = _att_scores(adj_ref[...], f1, f2_sc[...])
    att_ref[...] = att
    out = jnp.dot(att.astype(jnp.bfloat16), h_sc[...].astype(jnp.bfloat16),
                  preferred_element_type=jnp.float32)
    doc = jnp.where(out > 0, out, (jnp.exp(out) - 1.0)) \
        + x_ref[pl.ds(i * RBLK, RBLK), :]
    part = jnp.sum(doc, axis=0, keepdims=True)

    @pl.when(i == 0)
    def _():
        dsum_ref[...] = part

    @pl.when(i > 0)
    def _():
        dsum_ref[...] += part


_ROWBLK = pl.BlockSpec((RBLK, N), lambda i: (i, 0))
_FULL_X = pl.BlockSpec((N, WFEAT), lambda i: (0, 0))
_FULL_W = pl.BlockSpec((WFEAT, WFEAT), lambda i: (0, 0))
_FULL_A1 = pl.BlockSpec((WFEAT, 1), lambda i: (0, 0))
_FULL_A2 = pl.BlockSpec((1, WFEAT), lambda i: (0, 0))
_XBLK = pl.BlockSpec((RBLK, WFEAT), lambda i: (i, 0))
_ACC = pl.BlockSpec((1, WFEAT), lambda i: (0, 0))

_SCRATCH = [
    pltpu.VMEM((N, WFEAT), jnp.float32),
    pltpu.VMEM((N, 1), jnp.float32),
    pltpu.VMEM((1, N), jnp.float32),
]


def _gat_layer1(x, W, a1, a2, adj):
    return pl.pallas_call(
        _layer1_body,
        grid=(NBLK,),
        in_specs=[_FULL_X, _FULL_W, _FULL_A1, _FULL_A2, _ROWBLK],
        out_specs=(_ROWBLK, _XBLK),
        out_shape=(
            jax.ShapeDtypeStruct((N, N), jnp.float32),
            jax.ShapeDtypeStruct((N, WFEAT), jnp.float32),
        ),
        scratch_shapes=_SCRATCH,
        compiler_params=pltpu.CompilerParams(
            dimension_semantics=("arbitrary",)),
    )(x, W, a1, a2, adj)


def _gat_layer2(x, W, a1, a2, adj):
    return pl.pallas_call(
        _layer2_body,
        grid=(NBLK,),
        in_specs=[_FULL_X, _FULL_W, _FULL_A1, _FULL_A2, _ROWBLK],
        out_specs=(_ROWBLK, _ACC),
        out_shape=(
            jax.ShapeDtypeStruct((N, N), jnp.float32),
            jax.ShapeDtypeStruct((1, WFEAT), jnp.float32),
        ),
        scratch_shapes=_SCRATCH,
        compiler_params=pltpu.CompilerParams(
            dimension_semantics=("arbitrary",)),
    )(x, W, a1, a2, adj)


def kernel(inDoc, adj0, adj1, emb, W_s, a_s, W_d, a_d):
    words = _sc_gather(inDoc.astype(jnp.int32), emb)

    a1s = a_s[:WFEAT].reshape(WFEAT, 1)
    a2s = a_s[WFEAT:].reshape(1, WFEAT)
    satt, x2 = _gat_layer1(words, W_s, a1s, a2s, adj0.astype(jnp.int4))

    a1d = a_d[:WFEAT].reshape(WFEAT, 1)
    a2d = a_d[WFEAT:].reshape(1, WFEAT)
    datt, dsum = _gat_layer2(x2, W_d, a1d, a2d, adj1.astype(jnp.int4))

    docMean = (dsum / jnp.float32(N)).reshape(WFEAT)
    return (docMean, satt, datt)


# speedup vs baseline: 1.1118x; 1.0198x over previous
import functools

import jax
import jax.numpy as jnp
from jax import lax
from jax.experimental import pallas as pl
from jax.experimental.pallas import tpu as pltpu
from jax.experimental.pallas import tpu_sc as plsc

N = 4096
EDIM = 128
WFEAT = 128
SLOPE = 0.01
RBLK = 512
NBLK = N // RBLK
NEG = -1e9


def _make_sc_gather(V, D, B):
    info = plsc.get_sparse_core_info()
    NC, NS = info.num_cores, info.num_subcores
    NW = NC * NS
    assert B % (8 * NW) == 0 and D % info.num_lanes == 0
    b_per_w = B // NW
    mesh = plsc.VectorSubcoreMesh(core_axis_name="c", subcore_axis_name="s")

    @functools.partial(
        pl.kernel,
        mesh=mesh,
        out_type=jax.ShapeDtypeStruct((B, D), jnp.float32),
        scratch_types=[
            pltpu.VMEM((b_per_w,), jnp.int32),
            pltpu.VMEM((b_per_w, D), jnp.float32),
            pltpu.SemaphoreType.DMA,
        ],
    )
    def gather_k(idx_hbm, table_hbm, out_hbm, idx_v, rows_v, sem):
        wid = lax.axis_index("s") * NC + lax.axis_index("c")
        base = wid * b_per_w
        pltpu.sync_copy(idx_hbm.at[pl.ds(base, b_per_w)], idx_v)
        pltpu.async_copy(table_hbm.at[idx_v], rows_v, sem).wait()
        pltpu.sync_copy(rows_v, out_hbm.at[pl.ds(base, b_per_w)])

    return gather_k


@functools.lru_cache(maxsize=1)
def _sc_gather_fn():
    return _make_sc_gather(100000, EDIM, N)


def _sc_gather(idx, table):
    return _sc_gather_fn()(idx, table)


def _project(x, w_ref, a1_ref, a2_ref, h_sc, f1_sc, f2_sc):
    h = jnp.dot(x, w_ref[...], preferred_element_type=jnp.float32)
    h_sc[...] = h
    f1_sc[...] = jnp.dot(h, a1_ref[...], preferred_element_type=jnp.float32)
    f2_sc[...] = lax.dot_general(
        a2_ref[...], h, (((1,), (1,)), ((), ())),
        preferred_element_type=jnp.float32)


def _att_scores(adj, f1, f2row):
    s = f1 + f2row
    e = jnp.where(s >= 0, s, SLOPE * s)
    p = jnp.exp(jnp.where(adj.astype(jnp.int32) != 0, e, NEG))
    return p / jnp.sum(p, axis=1, keepdims=True)


def _elu(x):
    return jnp.where(x > 0, x, (jnp.exp(x) - 1.0))


def _body(x_ref, ws_ref, a1s_ref, a2s_ref, wd_ref, a1d_ref, a2d_ref, adj_ref,
          satt_ref, datt_ref, dsum_ref, h_sc, f1_sc, f2_sc, x2_sc):
    i = pl.program_id(0)

    @pl.when(i == 0)
    def _():
        _project(x_ref[...], ws_ref, a1s_ref, a2s_ref, h_sc, f1_sc, f2_sc)

    @pl.when(i == NBLK)
    def _():
        _project(x2_sc[...], wd_ref, a1d_ref, a2d_ref, h_sc, f1_sc, f2_sc)

    j = jnp.where(i < NBLK, i, i - NBLK)
    f1 = f1_sc[pl.ds(j * RBLK, RBLK), :]
    att = _att_scores(adj_ref[...], f1, f2_sc[...])
    out = jnp.dot(att.astype(jnp.bfloat16), h_sc[...].astype(jnp.bfloat16),
                  preferred_element_type=jnp.float32)

    @pl.when(i < NBLK)
    def _():
        satt_ref[...] = att
        x2_sc[pl.ds(i * RBLK, RBLK), :] = _elu(out)

    @pl.when(i >= NBLK)
    def _():
        datt_ref[...] = att

    doc = _elu(out) + x2_sc[pl.ds(j * RBLK, RBLK), :]
    part = jnp.sum(doc, axis=0, keepdims=True)

    @pl.when(i == NBLK)
    def _():
        dsum_ref[...] = part

    @pl.when(i > NBLK)
    def _():
        dsum_ref[...] += part


def _gat_both(x, W_s, a1s, a2s, W_d, a1d, a2d, adjc):
    full = lambda shape: pl.BlockSpec(shape, lambda i: (0, 0))
    return pl.pallas_call(
        _body,
        grid=(2 * NBLK,),
        in_specs=[
            full((N, WFEAT)),
            full((WFEAT, WFEAT)), full((WFEAT, 1)), full((1, WFEAT)),
            full((WFEAT, WFEAT)), full((WFEAT, 1)), full((1, WFEAT)),
            pl.BlockSpec((RBLK, N), lambda i: (i, 0)),
        ],
        out_specs=(
            pl.BlockSpec((RBLK, N), lambda i: (jnp.minimum(i, NBLK - 1), 0)),
            pl.BlockSpec((RBLK, N), lambda i: (jnp.maximum(i - NBLK, 0), 0)),
            pl.BlockSpec((1, WFEAT), lambda i: (0, 0)),
        ),
        out_shape=(
            jax.ShapeDtypeStruct((N, N), jnp.float32),
            jax.ShapeDtypeStruct((N, N), jnp.float32),
            jax.ShapeDtypeStruct((1, WFEAT), jnp.float32),
        ),
        scratch_shapes=[
            pltpu.VMEM((N, WFEAT), jnp.float32),
            pltpu.VMEM((N, 1), jnp.float32),
            pltpu.VMEM((1, N), jnp.float32),
            pltpu.VMEM((N, WFEAT), jnp.float32),
        ],
        compiler_params=pltpu.CompilerParams(
            dimension_semantics=("arbitrary",)),
    )(x, W_s, a1s, a2s, W_d, a1d, a2d, adjc)


def kernel(inDoc, adj0, adj1, emb, W_s, a_s, W_d, a_d):
    words = _sc_gather(inDoc.astype(jnp.int32), emb)
    adjc = jnp.concatenate([adj0, adj1], axis=0).astype(jnp.int4)

    satt, datt, dsum = _gat_both(
        words,
        W_s, a_s[:WFEAT].reshape(WFEAT, 1), a_s[WFEAT:].reshape(1, WFEAT),
        W_d, a_d[:WFEAT].reshape(WFEAT, 1), a_d[WFEAT:].reshape(1, WFEAT),
        adjc)

    docMean = (dsum / jnp.float32(N)).reshape(WFEAT)
    return (docMean, satt, datt)
